# baseline (speedup 1.0000x reference)
import functools

import jax
import jax.numpy as jnp
from jax.experimental import pallas as pl
from jax.experimental.pallas import tpu as pltpu

_LOG2E = 1.4426950408889634
_FAUG = 384
_BI = 256
_BJ = 4096


def _proj_kernel(x_ref, wv_ref, bv_ref, wq_ref, bq_ref, wk_ref, bk_ref,
                 xvb_ref, el_ref, er_ref, elmax_ref):
    i = pl.program_id(0)
    xv = jax.lax.dot_general(
        x_ref[...], wv_ref[...], (((1,), (1,)), ((), ())),
        preferred_element_type=jnp.float32) + bv_ref[...]
    xvb_ref[:, :xv.shape[1]] = xv.astype(jnp.bfloat16)
    lane = jax.lax.broadcasted_iota(
        jnp.int32, (xv.shape[0], _FAUG - xv.shape[1]), 1)
    xvb_ref[:, xv.shape[1]:] = (lane == 0).astype(jnp.bfloat16)
    el = jnp.sum(xv * wq_ref[...], axis=1, keepdims=True) + bq_ref[0, 0]
    el_ref[...] = el
    er_col = jnp.sum(xv * wk_ref[...], axis=1, keepdims=True) + bk_ref[0, 0]
    er_ref[...] = er_col.T
    bmax = jnp.max(el, keepdims=True)

    @pl.when(i == 0)
    def _():
        elmax_ref[...] = bmax

    @pl.when(i > 0)
    def _():
        elmax_ref[...] = jnp.maximum(elmax_ref[...], bmax)


def _attn_kernel(adj_ref, el_ref, er_ref, xvb_ref, elmax_ref, out_ref,
                 acc_ref, *, ni, f):
    i = pl.program_id(1)

    @pl.when(i == 0)
    def _():
        acc_ref[...] = jnp.zeros_like(acc_ref)

    el = el_ref[...]
    er = er_ref[...]
    mtop = elmax_ref[...] + er
    mj = jnp.maximum(mtop, 0.2 * mtop)
    a = (er - mj) * _LOG2E
    b = (0.2 * er - mj) * _LOG2E
    elc = el * _LOG2E
    elc2 = elc * 0.2
    z = jnp.maximum(elc + a, elc2 + b)
    p = (jnp.exp2(z) * adj_ref[...].astype(jnp.float32)).astype(jnp.bfloat16)
    acc_ref[...] += jax.lax.dot_general(
        p, xvb_ref[...], (((0,), (0,)), ((), ())),
        preferred_element_type=jnp.float32)

    @pl.when(i == ni - 1)
    def _():
        d = jnp.maximum(acc_ref[:, f:f + 1], 1e-20)
        out_ref[...] = acc_ref[:, :f] * (1.0 / d)


def kernel(x, adj, Wv, bv, wq, bq, wk, bk):
    n, _ = x.shape
    f = Wv.shape[0]

    bi1 = min(512, n)
    ni1 = n // bi1
    xvb, el, er, elmax = pl.pallas_call(
        _proj_kernel,
        grid=(ni1,),
        in_specs=[
            pl.BlockSpec((bi1, x.shape[1]), lambda i: (i, 0)),
            pl.BlockSpec(Wv.shape, lambda i: (0, 0)),
            pl.BlockSpec((1, f), lambda i: (0, 0)),
            pl.BlockSpec((1, f), lambda i: (0, 0)),
            pl.BlockSpec((1, 1), lambda i: (0, 0)),
            pl.BlockSpec((1, f), lambda i: (0, 0)),
            pl.BlockSpec((1, 1), lambda i: (0, 0)),
        ],
        out_specs=[
            pl.BlockSpec((bi1, _FAUG), lambda i: (i, 0)),
            pl.BlockSpec((bi1, 1), lambda i: (i, 0)),
            pl.BlockSpec((1, bi1), lambda i: (0, i)),
            pl.BlockSpec((1, 1), lambda i: (0, 0)),
        ],
        out_shape=[
            jax.ShapeDtypeStruct((n, _FAUG), jnp.bfloat16),
            jax.ShapeDtypeStruct((n, 1), jnp.float32),
            jax.ShapeDtypeStruct((1, n), jnp.float32),
            jax.ShapeDtypeStruct((1, 1), jnp.float32),
        ],
        compiler_params=pltpu.CompilerParams(
            dimension_semantics=("arbitrary",)),
    )(x, Wv, bv.reshape(1, f), wq, bq.reshape(1, 1), wk, bk.reshape(1, 1))

    bi = min(_BI, n)
    bj = min(_BJ, n)
    ni = n // bi
    nj = n // bj
    out = pl.pallas_call(
        functools.partial(_attn_kernel, ni=ni, f=f),
        grid=(nj, ni),
        in_specs=[
            pl.BlockSpec((bi, bj), lambda j, i: (i, j)),
            pl.BlockSpec((bi, 1), lambda j, i: (i, 0)),
            pl.BlockSpec((1, bj), lambda j, i: (0, j)),
            pl.BlockSpec((bi, _FAUG), lambda j, i: (i, 0)),
            pl.BlockSpec((1, 1), lambda j, i: (0, 0)),
        ],
        out_specs=pl.BlockSpec((bj, f), lambda j, i: (j, 0)),
        out_shape=jax.ShapeDtypeStruct((n, f), jnp.float32),
        scratch_shapes=[
            pltpu.VMEM((bj, _FAUG), jnp.float32),
        ],
        compiler_params=pltpu.CompilerParams(
            dimension_semantics=("parallel", "arbitrary")),
    )(adj, el, er, xvb, elmax)
    return out


# speedup vs baseline: 1.1020x; 1.1020x over previous
import functools

import jax
import jax.numpy as jnp
from jax.experimental import pallas as pl
from jax.experimental.pallas import tpu as pltpu

_LOG2E = 1.4426950408889634
_FAUG = 384
_BI = 512
_BJ = 4096


def _proj_kernel(x_ref, wv_ref, bv_ref, wq_ref, bq_ref, wk_ref, bk_ref,
                 xvb_ref, el_ref, er_ref, elmax_ref):
    i = pl.program_id(0)
    xv = jax.lax.dot_general(
        x_ref[...], wv_ref[...], (((1,), (1,)), ((), ())),
        preferred_element_type=jnp.float32) + bv_ref[...]
    xvb_ref[:, :xv.shape[1]] = xv.astype(jnp.bfloat16)
    lane = jax.lax.broadcasted_iota(
        jnp.int32, (xv.shape[0], _FAUG - xv.shape[1]), 1)
    xvb_ref[:, xv.shape[1]:] = (lane == 0).astype(jnp.bfloat16)
    el = jnp.sum(xv * wq_ref[...], axis=1, keepdims=True) + bq_ref[0, 0]
    el_ref[...] = el
    er_col = jnp.sum(xv * wk_ref[...], axis=1, keepdims=True) + bk_ref[0, 0]
    er_ref[...] = er_col.T
    bmax = jnp.max(el, keepdims=True)

    @pl.when(i == 0)
    def _():
        elmax_ref[...] = bmax

    @pl.when(i > 0)
    def _():
        elmax_ref[...] = jnp.maximum(elmax_ref[...], bmax)


def _attn_kernel(adj_ref, el_ref, er_ref, xvb_ref, elmax_ref, out_ref,
                 acc_ref, *, ni, f):
    i = pl.program_id(1)

    @pl.when(i == 0)
    def _():
        acc_ref[...] = jnp.zeros_like(acc_ref)

    el = el_ref[...]
    er = er_ref[...]
    mtop = elmax_ref[...] + er
    mj = jnp.maximum(mtop, 0.2 * mtop)
    a = (er - mj) * _LOG2E
    b = (0.2 * er - mj) * _LOG2E
    elc = el * _LOG2E
    elc2 = elc * 0.2
    z = jnp.maximum(elc + a, elc2 + b)
    p = (jnp.exp2(z) * adj_ref[...].astype(jnp.float32)).astype(jnp.bfloat16)
    acc_ref[...] += jax.lax.dot_general(
        p, xvb_ref[...], (((0,), (0,)), ((), ())),
        preferred_element_type=jnp.float32)

    @pl.when(i == ni - 1)
    def _():
        d = jnp.maximum(acc_ref[:, f:f + 1], 1e-20)
        out_ref[...] = acc_ref[:, :f] * (1.0 / d)


def kernel(x, adj, Wv, bv, wq, bq, wk, bk):
    n, _ = x.shape
    f = Wv.shape[0]

    bi1 = min(512, n)
    ni1 = n // bi1
    xvb, el, er, elmax = pl.pallas_call(
        _proj_kernel,
        grid=(ni1,),
        in_specs=[
            pl.BlockSpec((bi1, x.shape[1]), lambda i: (i, 0)),
            pl.BlockSpec(Wv.shape, lambda i: (0, 0)),
            pl.BlockSpec((1, f), lambda i: (0, 0)),
            pl.BlockSpec((1, f), lambda i: (0, 0)),
            pl.BlockSpec((1, 1), lambda i: (0, 0)),
            pl.BlockSpec((1, f), lambda i: (0, 0)),
            pl.BlockSpec((1, 1), lambda i: (0, 0)),
        ],
        out_specs=[
            pl.BlockSpec((bi1, _FAUG), lambda i: (i, 0)),
            pl.BlockSpec((bi1, 1), lambda i: (i, 0)),
            pl.BlockSpec((1, bi1), lambda i: (0, i)),
            pl.BlockSpec((1, 1), lambda i: (0, 0)),
        ],
        out_shape=[
            jax.ShapeDtypeStruct((n, _FAUG), jnp.bfloat16),
            jax.ShapeDtypeStruct((n, 1), jnp.float32),
            jax.ShapeDtypeStruct((1, n), jnp.float32),
            jax.ShapeDtypeStruct((1, 1), jnp.float32),
        ],
        compiler_params=pltpu.CompilerParams(
            dimension_semantics=("arbitrary",)),
    )(x, Wv, bv.reshape(1, f), wq, bq.reshape(1, 1), wk, bk.reshape(1, 1))

    bi = min(_BI, n)
    bj = min(_BJ, n)
    ni = n // bi
    nj = n // bj
    out = pl.pallas_call(
        functools.partial(_attn_kernel, ni=ni, f=f),
        grid=(nj, ni),
        in_specs=[
            pl.BlockSpec((bi, bj), lambda j, i: (i, j)),
            pl.BlockSpec((bi, 1), lambda j, i: (i, 0)),
            pl.BlockSpec((1, bj), lambda j, i: (0, j)),
            pl.BlockSpec((bi, _FAUG), lambda j, i: (i, 0)),
            pl.BlockSpec((1, 1), lambda j, i: (0, 0)),
        ],
        out_specs=pl.BlockSpec((bj, f), lambda j, i: (j, 0)),
        out_shape=jax.ShapeDtypeStruct((n, f), jnp.float32),
        scratch_shapes=[
            pltpu.VMEM((bj, _FAUG), jnp.float32),
        ],
        compiler_params=pltpu.CompilerParams(
            dimension_semantics=("parallel", "arbitrary")),
    )(adj, el, er, xvb, elmax)
    return out


# speedup vs baseline: 1.2108x; 1.0987x over previous
import functools

import jax
import jax.numpy as jnp
from jax.experimental import pallas as pl
from jax.experimental.pallas import tpu as pltpu

_LOG2E = 1.4426950408889634
_FAUG = 384
_BI = 512


def _fused_kernel(x_ref, wv_ref, bv_ref, wq_ref, bq_ref, wk_ref, bk_ref,
                  adj_ref, out_ref,
                  xvb_ref, elc_ref, ercol_ref, a_ref, b_ref, elmax_ref,
                  acc_ref, *, ni1, ni, bi1, bi, f):
    s = pl.program_id(0)

    @pl.when(s < ni1)
    def _proj():
        xv = jax.lax.dot_general(
            x_ref[...], wv_ref[...], (((1,), (1,)), ((), ())),
            preferred_element_type=jnp.float32) + bv_ref[...]
        row0 = s * bi1
        xvb_ref[pl.ds(row0, bi1), :f] = xv.astype(jnp.bfloat16)
        lane = jax.lax.broadcasted_iota(jnp.int32, (bi1, _FAUG - f), 1)
        xvb_ref[pl.ds(row0, bi1), f:] = (lane == 0).astype(jnp.bfloat16)
        el = jnp.sum(xv * wq_ref[...], axis=1, keepdims=True) + bq_ref[0, 0]
        elc_ref[pl.ds(row0, bi1), :] = el * _LOG2E
        er = jnp.sum(xv * wk_ref[...], axis=1, keepdims=True) + bk_ref[0, 0]
        ercol_ref[pl.ds(row0, bi1), :] = er
        bmax = jnp.max(el, keepdims=True)

        @pl.when(s == 0)
        def _():
            elmax_ref[...] = bmax

        @pl.when(s > 0)
        def _():
            elmax_ref[...] = jnp.maximum(elmax_ref[...], bmax)

    @pl.when(s == ni1)
    def _setup():
        er_row = ercol_ref[...].T
        mtop = elmax_ref[...] + er_row
        mj = jnp.maximum(mtop, 0.2 * mtop)
        a_ref[...] = (er_row - mj) * _LOG2E
        b_ref[...] = (0.2 * er_row - mj) * _LOG2E
        acc_ref[...] = jnp.zeros_like(acc_ref)

    @pl.when(s >= ni1)
    def _attn():
        i = s - ni1
        row0 = i * bi
        elc = elc_ref[pl.ds(row0, bi), :]
        elc2 = elc * 0.2
        z = jnp.maximum(elc + a_ref[...], elc2 + b_ref[...])
        p = (jnp.exp2(z)
             * adj_ref[...].astype(jnp.float32)).astype(jnp.bfloat16)
        acc_ref[...] += jax.lax.dot_general(
            p, xvb_ref[pl.ds(row0, bi), :], (((0,), (0,)), ((), ())),
            preferred_element_type=jnp.float32)

        @pl.when(s == ni1 + ni - 1)
        def _final():
            d = jnp.maximum(acc_ref[:, f:f + 1], 1e-20)
            out_ref[...] = acc_ref[:, :f] * (1.0 / d)


def kernel(x, adj, Wv, bv, wq, bq, wk, bk):
    n, _ = x.shape
    f = Wv.shape[0]

    bi1 = min(512, n)
    ni1 = n // bi1
    bi = min(_BI, n)
    ni = n // bi

    out = pl.pallas_call(
        functools.partial(_fused_kernel, ni1=ni1, ni=ni, bi1=bi1, bi=bi, f=f),
        grid=(ni1 + ni,),
        in_specs=[
            pl.BlockSpec((bi1, x.shape[1]),
                         lambda s: (jnp.minimum(s, ni1 - 1), 0)),
            pl.BlockSpec(Wv.shape, lambda s: (0, 0)),
            pl.BlockSpec((1, f), lambda s: (0, 0)),
            pl.BlockSpec((1, f), lambda s: (0, 0)),
            pl.BlockSpec((1, 1), lambda s: (0, 0)),
            pl.BlockSpec((1, f), lambda s: (0, 0)),
            pl.BlockSpec((1, 1), lambda s: (0, 0)),
            pl.BlockSpec((bi, n), lambda s: (jnp.maximum(s - ni1, 0), 0)),
        ],
        out_specs=pl.BlockSpec((n, f), lambda s: (0, 0)),
        out_shape=jax.ShapeDtypeStruct((n, f), jnp.float32),
        scratch_shapes=[
            pltpu.VMEM((n, _FAUG), jnp.bfloat16),
            pltpu.VMEM((n, 1), jnp.float32),
            pltpu.VMEM((n, 1), jnp.float32),
            pltpu.VMEM((1, n), jnp.float32),
            pltpu.VMEM((1, n), jnp.float32),
            pltpu.VMEM((1, 1), jnp.float32),
            pltpu.VMEM((n, _FAUG), jnp.float32),
        ],
        compiler_params=pltpu.CompilerParams(
            dimension_semantics=("arbitrary",)),
    )(x, Wv, bv.reshape(1, f), wq, bq.reshape(1, 1), wk, bk.reshape(1, 1),
      adj)
    return out


# speedup vs baseline: 1.2120x; 1.0010x over previous
import functools

import jax
import jax.numpy as jnp
from jax.experimental import pallas as pl
from jax.experimental.pallas import tpu as pltpu

_LOG2E = 1.4426950408889634
_FAUG = 384
_BI = 512


def _fused_kernel(x_ref, wv_ref, bv_ref, wq_ref, bq_ref, wk_ref, bk_ref,
                  adj_ref, adj2_ref, out_ref,
                  xvb_ref, elc_ref, ercol_ref, a_ref, b_ref, elmax_ref,
                  acc_ref, *, ni1, ni, bi1, bi, f):
    s = pl.program_id(0)

    @pl.when(s < ni1)
    def _proj():
        xv = jax.lax.dot_general(
            x_ref[...], wv_ref[...], (((1,), (1,)), ((), ())),
            preferred_element_type=jnp.float32) + bv_ref[...]
        row0 = s * bi1
        xvb_ref[pl.ds(row0, bi1), :f] = xv.astype(jnp.bfloat16)
        lane = jax.lax.broadcasted_iota(jnp.int32, (bi1, _FAUG - f), 1)
        xvb_ref[pl.ds(row0, bi1), f:] = (lane == 0).astype(jnp.bfloat16)
        el = jnp.sum(xv * wq_ref[...], axis=1, keepdims=True) + bq_ref[0, 0]
        elc_ref[pl.ds(row0, bi1), :] = el * _LOG2E
        er = jnp.sum(xv * wk_ref[...], axis=1, keepdims=True) + bk_ref[0, 0]
        ercol_ref[pl.ds(row0, bi1), :] = er
        bmax = jnp.max(el, keepdims=True)

        @pl.when(s == 0)
        def _():
            elmax_ref[...] = bmax

        @pl.when(s > 0)
        def _():
            elmax_ref[...] = jnp.maximum(elmax_ref[...], bmax)

    @pl.when(s == ni1)
    def _setup():
        er_row = ercol_ref[...].T
        mtop = elmax_ref[...] + er_row
        mj = jnp.maximum(mtop, 0.2 * mtop)
        a_ref[...] = (er_row - mj) * _LOG2E
        b_ref[...] = (0.2 * er_row - mj) * _LOG2E
        acc_ref[...] = jnp.zeros_like(acc_ref)

    @pl.when(s >= ni1)
    def _attn():
        i = s - ni1
        hb = bi // 2
        a = a_ref[...]
        b = b_ref[...]

        def half_product(adj_half_ref, row0):
            elc = elc_ref[pl.ds(row0, hb), :]
            elc2 = elc * 0.2
            z = jnp.maximum(elc + a, elc2 + b)
            p = (jnp.exp2(z)
                 * adj_half_ref[...].astype(jnp.float32)).astype(jnp.bfloat16)
            return jax.lax.dot_general(
                p, xvb_ref[pl.ds(row0, hb), :], (((0,), (0,)), ((), ())),
                preferred_element_type=jnp.float32)

        acc_ref[...] += (half_product(adj_ref, i * bi)
                         + half_product(adj2_ref, i * bi + hb))

        @pl.when(s == ni1 + ni - 1)
        def _final():
            d = jnp.maximum(acc_ref[:, f:f + 1], 1e-20)
            out_ref[...] = acc_ref[:, :f] * (1.0 / d)


def kernel(x, adj, Wv, bv, wq, bq, wk, bk):
    n, _ = x.shape
    f = Wv.shape[0]

    bi1 = min(512, n)
    ni1 = n // bi1
    bi = min(_BI, n)
    ni = n // bi

    out = pl.pallas_call(
        functools.partial(_fused_kernel, ni1=ni1, ni=ni, bi1=bi1, bi=bi, f=f),
        grid=(ni1 + ni,),
        in_specs=[
            pl.BlockSpec((bi1, x.shape[1]),
                         lambda s: (jnp.minimum(s, ni1 - 1), 0)),
            pl.BlockSpec(Wv.shape, lambda s: (0, 0)),
            pl.BlockSpec((1, f), lambda s: (0, 0)),
            pl.BlockSpec((1, f), lambda s: (0, 0)),
            pl.BlockSpec((1, 1), lambda s: (0, 0)),
            pl.BlockSpec((1, f), lambda s: (0, 0)),
            pl.BlockSpec((1, 1), lambda s: (0, 0)),
            pl.BlockSpec((bi // 2, n),
                         lambda s: (2 * jnp.maximum(s - ni1, 0), 0)),
            pl.BlockSpec((bi // 2, n),
                         lambda s: (2 * jnp.maximum(s - ni1, 0) + 1, 0)),
        ],
        out_specs=pl.BlockSpec((n, f), lambda s: (0, 0)),
        out_shape=jax.ShapeDtypeStruct((n, f), jnp.float32),
        scratch_shapes=[
            pltpu.VMEM((n, _FAUG), jnp.bfloat16),
            pltpu.VMEM((n, 1), jnp.float32),
            pltpu.VMEM((n, 1), jnp.float32),
            pltpu.VMEM((1, n), jnp.float32),
            pltpu.VMEM((1, n), jnp.float32),
            pltpu.VMEM((1, 1), jnp.float32),
            pltpu.VMEM((n, _FAUG), jnp.float32),
        ],
        compiler_params=pltpu.CompilerParams(
            dimension_semantics=("arbitrary",)),
    )(x, Wv, bv.reshape(1, f), wq, bq.reshape(1, 1), wk, bk.reshape(1, 1),
      adj, adj)
    return out


# speedup vs baseline: 1.3685x; 1.1291x over previous
import functools

import jax
import jax.numpy as jnp
from jax.experimental import pallas as pl
from jax.experimental.pallas import tpu as pltpu

_LOG2E = 1.4426950408889634
_FAUG = 384
_BI = 512


def _fused_kernel(x_ref, wv_ref, bv_ref, wq_ref, bq_ref, wk_ref, bk_ref,
                  adj_ref, adj2_ref, out_ref,
                  xvb_ref, elc_ref, ercol_ref, a_ref, b_ref, elmax_ref,
                  acc_ref, *, ni1, ni, bi1, bi, f):
    s = pl.program_id(0)

    @pl.when(s < ni1)
    def _proj():
        xv = jax.lax.dot_general(
            x_ref[...], wv_ref[...], (((1,), (1,)), ((), ())),
            preferred_element_type=jnp.float32) + bv_ref[...]
        row0 = s * bi1
        xvb_ref[pl.ds(row0, bi1), :f] = xv.astype(jnp.bfloat16)
        lane = jax.lax.broadcasted_iota(jnp.int32, (bi1, _FAUG - f), 1)
        xvb_ref[pl.ds(row0, bi1), f:] = (lane == 0).astype(jnp.bfloat16)
        el = jnp.sum(xv * wq_ref[...], axis=1, keepdims=True) + bq_ref[0, 0]
        elc_ref[pl.ds(row0, bi1), :] = el * _LOG2E
        er = jnp.sum(xv * wk_ref[...], axis=1, keepdims=True) + bk_ref[0, 0]
        ercol_ref[pl.ds(row0, bi1), :] = er
        bmax = jnp.max(el, keepdims=True)

        @pl.when(s == 0)
        def _():
            elmax_ref[...] = bmax

        @pl.when(s > 0)
        def _():
            elmax_ref[...] = jnp.maximum(elmax_ref[...], bmax)

    @pl.when(s == ni1)
    def _setup():
        er_row = ercol_ref[...].T
        mtop = elmax_ref[...] + er_row
        mj = jnp.maximum(mtop, 0.2 * mtop)
        a_ref[...] = (er_row - mj) * _LOG2E
        b_ref[...] = (0.2 * er_row - mj) * _LOG2E
        acc_ref[...] = jnp.zeros_like(acc_ref)

    @pl.when(s >= ni1)
    def _attn():
        i = s - ni1
        hb = bi // 2
        a = a_ref[...].astype(jnp.bfloat16)
        b = b_ref[...].astype(jnp.bfloat16)

        def half_product(adj_half_ref, row0):
            elc = elc_ref[pl.ds(row0, hb), :].astype(jnp.bfloat16)
            elc2 = elc * jnp.bfloat16(0.2)
            z = jnp.maximum(elc + a, elc2 + b)
            p = jnp.exp2(z) * adj_half_ref[...].astype(jnp.bfloat16)
            return jax.lax.dot_general(
                p, xvb_ref[pl.ds(row0, hb), :], (((0,), (0,)), ((), ())),
                preferred_element_type=jnp.float32)

        acc_ref[...] += (half_product(adj_ref, i * bi)
                         + half_product(adj2_ref, i * bi + hb))

        @pl.when(s == ni1 + ni - 1)
        def _final():
            d = jnp.maximum(acc_ref[:, f:f + 1], 1e-20)
            out_ref[...] = acc_ref[:, :f] * (1.0 / d)


def kernel(x, adj, Wv, bv, wq, bq, wk, bk):
    n, _ = x.shape
    f = Wv.shape[0]

    bi1 = min(1024, n)
    ni1 = n // bi1
    bi = min(_BI, n)
    ni = n // bi

    out = pl.pallas_call(
        functools.partial(_fused_kernel, ni1=ni1, ni=ni, bi1=bi1, bi=bi, f=f),
        grid=(ni1 + ni,),
        in_specs=[
            pl.BlockSpec((bi1, x.shape[1]),
                         lambda s: (jnp.minimum(s, ni1 - 1), 0)),
            pl.BlockSpec(Wv.shape, lambda s: (0, 0)),
            pl.BlockSpec((1, f), lambda s: (0, 0)),
            pl.BlockSpec((1, f), lambda s: (0, 0)),
            pl.BlockSpec((1, 1), lambda s: (0, 0)),
            pl.BlockSpec((1, f), lambda s: (0, 0)),
            pl.BlockSpec((1, 1), lambda s: (0, 0)),
            pl.BlockSpec((bi // 2, n),
                         lambda s: (2 * jnp.maximum(s - ni1, 0), 0)),
            pl.BlockSpec((bi // 2, n),
                         lambda s: (2 * jnp.maximum(s - ni1, 0) + 1, 0)),
        ],
        out_specs=pl.BlockSpec((n, f), lambda s: (0, 0)),
        out_shape=jax.ShapeDtypeStruct((n, f), jnp.float32),
        scratch_shapes=[
            pltpu.VMEM((n, _FAUG), jnp.bfloat16),
            pltpu.VMEM((n, 1), jnp.float32),
            pltpu.VMEM((n, 1), jnp.float32),
            pltpu.VMEM((1, n), jnp.float32),
            pltpu.VMEM((1, n), jnp.float32),
            pltpu.VMEM((1, 1), jnp.float32),
            pltpu.VMEM((n, _FAUG), jnp.float32),
        ],
        compiler_params=pltpu.CompilerParams(
            dimension_semantics=("arbitrary",)),
    )(x, Wv, bv.reshape(1, f), wq, bq.reshape(1, 1), wk, bk.reshape(1, 1),
      adj, adj)
    return out
